# baseline (speedup 1.0000x reference)
import functools

import jax
import jax.numpy as jnp
from jax import lax
from jax.experimental import pallas as pl
from jax.experimental.pallas import tpu as pltpu
from jax.experimental.pallas import tpu_sc as plsc

N = 100000
D = 128
B = 512
P = D // 16

NC = 2
NS = 16
NW = NC * NS
L = 16

SEG_PER_W = B // NW
ROWS = 384
NEG_INF = float("-inf")


def _body(feats_hbm, ids_hbm, w_hbm, b_hbm, out_hbm,
          fbuf, idsbuf, acc, wbuf, bbuf, idxa, idxb, vala, valb,
          offsa, offsb, sem, sem2, semf0, semf1):
    cid = lax.axis_index("c")
    sid = lax.axis_index("s")
    wid = sid * NC + cid
    seg_base = SEG_PER_W * wid

    pltpu.sync_copy(w_hbm, wbuf)
    pltpu.sync_copy(b_hbm, bbuf)
    wv = [wbuf[pl.ds(16 * p, 16)] for p in range(P)]
    bval = bbuf[...][0]

    zero = jnp.zeros((16,), jnp.float32)
    ninf = jnp.full((16,), NEG_INF, jnp.float32)
    for k in range(SEG_PER_W):
        for p in range(P):
            acc[k, pl.ds(16 * p, 16)] = zero
            acc[k, pl.ds(D + 16 * p, 16)] = ninf

    io = lax.iota(jnp.int32, 16)
    ta = seg_base + io
    tb = seg_base + 1 + io
    zi = jnp.zeros((16,), jnp.int32)
    nn = jnp.full((16,), N, jnp.int32)

    def search_step(_, carry):
        loa, hia, lob, hib = carry
        mida = lax.shift_right_logical(loa + hia, 1)
        midb = lax.shift_right_logical(lob + hib, 1)
        idxa[...] = jnp.minimum(mida, N - 1)
        idxb[...] = jnp.minimum(midb, N - 1)
        cpa = pltpu.async_copy(ids_hbm.at[idxa], vala, sem)
        cpb = pltpu.async_copy(ids_hbm.at[idxb], valb, sem2)
        cpa.wait()
        cpb.wait()
        aa = loa < hia
        pa = vala[...] >= ta
        ab = lob < hib
        pb = valb[...] >= tb
        hia = jnp.where(aa & pa, mida, hia)
        loa = jnp.where(aa & (~pa), mida + 1, loa)
        hib = jnp.where(ab & pb, midb, hib)
        lob = jnp.where(ab & (~pb), midb + 1, lob)
        return loa, hia, lob, hib

    starts, _, ends, _ = lax.fori_loop(
        0, 17, search_step, (zi, nn, zi, nn))
    idxa[...] = starts
    starts, _, ends, _ = lax.fori_loop(
        0, 17, search_step, (zi, nn, zi, nn))
    offsa[pl.ds(0, 16)] = starts
    offsb[pl.ds(0, 16)] = ends
    r0 = starts[0]
    r1 = ends[15]

    a0 = (r0 // 8) * 8
    nchunks = lax.div(r1 - a0 + (ROWS - 1), ROWS)
    semf = [semf0, semf1]

    def chunk_base(c):
        return jnp.minimum(a0 + c * ROWS, N - ROWS)

    IW = ROWS + 16

    def issue(c, nb):
        base = chunk_base(c)
        pltpu.async_copy(feats_hbm.at[pl.ds(base, ROWS)],
                         fbuf.at[pl.ds(nb * ROWS, ROWS)], semf[nb])
        pltpu.async_copy(ids_hbm.at[pl.ds(base, ROWS)],
                         idsbuf.at[pl.ds(nb * IW, ROWS)], semf[nb])

    def wait(c, nb):
        base = chunk_base(c)
        pltpu.make_async_copy(feats_hbm.at[pl.ds(base, ROWS)],
                              fbuf.at[pl.ds(nb * ROWS, ROWS)],
                              semf[nb]).wait()
        pltpu.make_async_copy(ids_hbm.at[pl.ds(base, ROWS)],
                              idsbuf.at[pl.ds(nb * IW, ROWS)],
                              semf[nb]).wait()

    def process(c, nb):
        cs = a0 + c * ROWS
        base = chunk_base(c)
        g0 = jnp.maximum(r0, cs)
        g1 = jnp.minimum(cs + ROWS, r1)

        @pl.when(g1 > g0)
        def _process():
            kf = idsbuf[pl.ds(nb * IW + g0 - base, 16)][0] - seg_base
            kl = idsbuf[pl.ds(nb * IW + g1 - 1 - base, 16)][0] - seg_base

            def do_seg(k, _):
                s0 = jnp.maximum(offsa[pl.ds(k, 16)][0], g0)
                s1 = jnp.minimum(offsb[pl.ds(k, 16)][0], g1)

                @plsc.parallel_loop(
                    s0 - base, s1 - base, unroll=4,
                    carry=(tuple(zero for _ in range(P)),
                           tuple(ninf for _ in range(P))))
                def rowloop(j, carry):
                    ss, mm = carry
                    rp = [fbuf[nb * ROWS + j, pl.ds(16 * p, 16)]
                          for p in range(P)]
                    dv = rp[0] * wv[0]
                    for p in range(1, P):
                        dv = dv + rp[p] * wv[p]
                    dot = lax.reduce_sum_p.bind(dv, axes=(0,)) + bval
                    dotv = jnp.full((16,), dot, jnp.float32)
                    g = 1.0 / (1.0 + jnp.exp(-dotv))
                    ss = tuple(ss[p] + rp[p] * g for p in range(P))
                    mm = tuple(jnp.maximum(mm[p], rp[p]) for p in range(P))
                    return ss, mm

                ss, mm = rowloop
                for p in range(P):
                    plsc.addupdate(acc.at[k, pl.ds(16 * p, 16)], ss[p])
                    mo = acc[k, pl.ds(D + 16 * p, 16)]
                    acc[k, pl.ds(D + 16 * p, 16)] = jnp.maximum(mo, mm[p])
                return 0

            lax.fori_loop(kf, kl + 1, do_seg, 0)

    issue(0, 0)

    def do_pair(pair, _):
        cc = 2 * pair
        for b2 in range(2):
            c = cc + b2

            @pl.when(c < nchunks)
            def _step():
                @pl.when(c + 1 < nchunks)
                def _prefetch():
                    issue(c + 1, (b2 + 1) % 2)

                wait(c, b2)
                process(c, b2)
        return 0

    lax.fori_loop(0, lax.div(nchunks + 1, 2), do_pair, 0)

    pltpu.sync_copy(acc, out_hbm.at[pl.ds(seg_base, SEG_PER_W)])


@jax.jit
def _run(feats, segment_ids, wvec, bpad):
    mesh = plsc.VectorSubcoreMesh(
        core_axis_name="c", subcore_axis_name="s",
        num_cores=NC, num_subcores=NS)
    return pl.kernel(
        _body,
        out_type=jax.ShapeDtypeStruct((B, 2 * D), jnp.float32),
        mesh=mesh,
        compiler_params=pltpu.CompilerParams(needs_layout_passes=False),
        scratch_types=[
            pltpu.VMEM((2 * ROWS, D), jnp.float32),
            pltpu.VMEM((2 * (ROWS + 16),), jnp.int32),
            pltpu.VMEM((SEG_PER_W, 2 * D), jnp.float32),
            pltpu.VMEM((D,), jnp.float32),
            pltpu.VMEM((16,), jnp.float32),
            pltpu.VMEM((16,), jnp.int32),
            pltpu.VMEM((16,), jnp.int32),
            pltpu.VMEM((16,), jnp.int32),
            pltpu.VMEM((16,), jnp.int32),
            pltpu.VMEM((32,), jnp.int32),
            pltpu.VMEM((32,), jnp.int32),
            pltpu.SemaphoreType.DMA,
            pltpu.SemaphoreType.DMA,
            pltpu.SemaphoreType.DMA,
            pltpu.SemaphoreType.DMA,
        ],
    )(feats, segment_ids, wvec, bpad)


def kernel(feats, segment_ids, W, b):
    wvec = W.reshape(D)
    bpad = jnp.broadcast_to(b.reshape(1), (16,)).astype(jnp.float32)
    return _run(feats, segment_ids.astype(jnp.int32), wvec, bpad)


# speedup vs baseline: 1.5110x; 1.5110x over previous
import functools

import jax
import jax.numpy as jnp
from jax import lax
from jax.experimental import pallas as pl
from jax.experimental.pallas import tpu as pltpu
from jax.experimental.pallas import tpu_sc as plsc

N = 100000
D = 128
B = 512
P = D // 16

NC = 2
NS = 16
NW = NC * NS
L = 16

SEG_PER_W = B // NW
ROWS = 384
NEG_INF = float("-inf")


def _body(feats_hbm, ids_hbm, w_hbm, b_hbm, out_hbm,
          fbuf, idsbuf, acc, wbuf, bbuf, idxa, idxb, vala, valb,
          offsa, offsb, gatebuf, sem, sem2, semf0, semf1):
    cid = lax.axis_index("c")
    sid = lax.axis_index("s")
    wid = sid * NC + cid
    seg_base = SEG_PER_W * wid

    pltpu.sync_copy(w_hbm, wbuf)
    pltpu.sync_copy(b_hbm, bbuf)
    wv = [wbuf[pl.ds(16 * p, 16)] for p in range(P)]
    bval = bbuf[...][0]

    zero = jnp.zeros((16,), jnp.float32)
    ninf = jnp.full((16,), NEG_INF, jnp.float32)
    for k in range(SEG_PER_W):
        for p in range(P):
            acc[k, pl.ds(16 * p, 16)] = zero
            acc[k, pl.ds(D + 16 * p, 16)] = ninf

    io = lax.iota(jnp.int32, 16)
    ta = seg_base + io
    tb = seg_base + 1 + io
    zi = jnp.zeros((16,), jnp.int32)
    nn = jnp.full((16,), N, jnp.int32)

    def search_step(_, carry):
        loa, hia, lob, hib = carry
        mida = lax.shift_right_logical(loa + hia, 1)
        midb = lax.shift_right_logical(lob + hib, 1)
        idxa[...] = jnp.minimum(mida, N - 1)
        idxb[...] = jnp.minimum(midb, N - 1)
        cpa = pltpu.async_copy(ids_hbm.at[idxa], vala, sem)
        cpb = pltpu.async_copy(ids_hbm.at[idxb], valb, sem2)
        cpa.wait()
        cpb.wait()
        aa = loa < hia
        pa = vala[...] >= ta
        ab = lob < hib
        pb = valb[...] >= tb
        hia = jnp.where(aa & pa, mida, hia)
        loa = jnp.where(aa & (~pa), mida + 1, loa)
        hib = jnp.where(ab & pb, midb, hib)
        lob = jnp.where(ab & (~pb), midb + 1, lob)
        return loa, hia, lob, hib

    starts, _, ends, _ = lax.fori_loop(
        0, 17, search_step, (zi, nn, zi, nn))
    offsa[pl.ds(0, 16)] = starts
    offsb[pl.ds(0, 16)] = ends
    r0 = starts[0]
    r1 = ends[15]

    a0 = (r0 // 8) * 8
    nchunks = lax.div(r1 - a0 + (ROWS - 1), ROWS)
    semf = [semf0, semf1]

    def chunk_base(c):
        return jnp.minimum(a0 + c * ROWS, N - ROWS)

    IW = ROWS + 16

    def issue(c, nb):
        base = chunk_base(c)
        pltpu.async_copy(feats_hbm.at[pl.ds(base, ROWS)],
                         fbuf.at[pl.ds(nb * ROWS, ROWS)], semf[nb])
        pltpu.async_copy(ids_hbm.at[pl.ds(base, ROWS)],
                         idsbuf.at[pl.ds(nb * IW, ROWS)], semf[nb])

    def wait(c, nb):
        base = chunk_base(c)
        pltpu.make_async_copy(feats_hbm.at[pl.ds(base, ROWS)],
                              fbuf.at[pl.ds(nb * ROWS, ROWS)],
                              semf[nb]).wait()
        pltpu.make_async_copy(ids_hbm.at[pl.ds(base, ROWS)],
                              idsbuf.at[pl.ds(nb * IW, ROWS)],
                              semf[nb]).wait()

    def process(c, nb):
        cs = a0 + c * ROWS
        base = chunk_base(c)
        g0 = jnp.maximum(r0, cs)
        g1 = jnp.minimum(cs + ROWS, r1)

        @pl.when(g1 > g0)
        def _process():
            jlo = lax.div(g0 - base, 16)
            jhi = lax.div(g1 - base + 15, 16)

            def gloop(jj, _):
                gv = zero
                for u in range(16):
                    rp = [fbuf[nb * ROWS + 16 * jj + u, pl.ds(16 * p, 16)]
                          for p in range(P)]
                    dv = rp[0] * wv[0]
                    for p in range(1, P):
                        dv = dv + rp[p] * wv[p]
                    dot = lax.reduce_sum_p.bind(dv, axes=(0,)) + bval
                    gv = jnp.where(io == u, jnp.full((16,), dot), gv)
                gatebuf[pl.ds(16 * jj, 16)] = 1.0 / (1.0 + jnp.exp(-gv))
                return 0

            lax.fori_loop(jlo, jhi, gloop, 0)

            kf = idsbuf[pl.ds(nb * IW + g0 - base, 16)][0] - seg_base
            kl = idsbuf[pl.ds(nb * IW + g1 - 1 - base, 16)][0] - seg_base

            def do_seg(k, _):
                s0 = jnp.maximum(offsa[pl.ds(k, 16)][0], g0)
                s1 = jnp.minimum(offsb[pl.ds(k, 16)][0], g1)

                @plsc.parallel_loop(
                    s0 - base, s1 - base, unroll=4,
                    carry=(tuple(zero for _ in range(P)),
                           tuple(ninf for _ in range(P))))
                def rowloop(j, carry):
                    ss, mm = carry
                    g = jnp.full((16,), gatebuf[pl.ds(j, 16)][0],
                                 jnp.float32)
                    rp = [fbuf[nb * ROWS + j, pl.ds(16 * p, 16)]
                          for p in range(P)]
                    ss = tuple(ss[p] + rp[p] * g for p in range(P))
                    mm = tuple(jnp.maximum(mm[p], rp[p]) for p in range(P))
                    return ss, mm

                ss, mm = rowloop
                for p in range(P):
                    plsc.addupdate(acc.at[k, pl.ds(16 * p, 16)], ss[p])
                    mo = acc[k, pl.ds(D + 16 * p, 16)]
                    acc[k, pl.ds(D + 16 * p, 16)] = jnp.maximum(mo, mm[p])
                return 0

            lax.fori_loop(kf, kl + 1, do_seg, 0)

    issue(0, 0)

    def do_pair(pair, _):
        cc = 2 * pair
        for b2 in range(2):
            c = cc + b2

            @pl.when(c < nchunks)
            def _step():
                @pl.when(c + 1 < nchunks)
                def _prefetch():
                    issue(c + 1, (b2 + 1) % 2)

                wait(c, b2)
                process(c, b2)
        return 0

    lax.fori_loop(0, lax.div(nchunks + 1, 2), do_pair, 0)

    pltpu.sync_copy(acc, out_hbm.at[pl.ds(seg_base, SEG_PER_W)])


@jax.jit
def _run(feats, segment_ids, wvec, bpad):
    mesh = plsc.VectorSubcoreMesh(
        core_axis_name="c", subcore_axis_name="s",
        num_cores=NC, num_subcores=NS)
    return pl.kernel(
        _body,
        out_type=jax.ShapeDtypeStruct((B, 2 * D), jnp.float32),
        mesh=mesh,
        compiler_params=pltpu.CompilerParams(needs_layout_passes=False),
        scratch_types=[
            pltpu.VMEM((2 * ROWS, D), jnp.float32),
            pltpu.VMEM((2 * (ROWS + 16),), jnp.int32),
            pltpu.VMEM((SEG_PER_W, 2 * D), jnp.float32),
            pltpu.VMEM((D,), jnp.float32),
            pltpu.VMEM((16,), jnp.float32),
            pltpu.VMEM((16,), jnp.int32),
            pltpu.VMEM((16,), jnp.int32),
            pltpu.VMEM((16,), jnp.int32),
            pltpu.VMEM((16,), jnp.int32),
            pltpu.VMEM((32,), jnp.int32),
            pltpu.VMEM((32,), jnp.int32),
            pltpu.VMEM((ROWS + 16,), jnp.float32),
            pltpu.SemaphoreType.DMA,
            pltpu.SemaphoreType.DMA,
            pltpu.SemaphoreType.DMA,
            pltpu.SemaphoreType.DMA,
        ],
    )(feats, segment_ids, wvec, bpad)


def kernel(feats, segment_ids, W, b):
    wvec = W.reshape(D)
    bpad = jnp.broadcast_to(b.reshape(1), (16,)).astype(jnp.float32)
    return _run(feats, segment_ids.astype(jnp.int32), wvec, bpad)


# speedup vs baseline: 1.5370x; 1.0172x over previous
import functools

import jax
import jax.numpy as jnp
from jax import lax
from jax.experimental import pallas as pl
from jax.experimental.pallas import tpu as pltpu
from jax.experimental.pallas import tpu_sc as plsc

N = 100000
D = 128
B = 512
P = D // 16

NC = 2
NS = 16
NW = NC * NS
L = 16

SEG_PER_W = B // NW
ROWS = 384
NEG_INF = float("-inf")


def _body(feats_hbm, ids_hbm, w_hbm, b_hbm, out_hbm,
          fbuf, idsbuf, acc, wbuf, bbuf, idxa, idxb, vala, valb,
          offsa, offsb, gatebuf, sem, sem2, semf0, semf1):
    cid = lax.axis_index("c")
    sid = lax.axis_index("s")
    wid = sid * NC + cid
    seg_base = SEG_PER_W * wid

    pltpu.sync_copy(w_hbm, wbuf)
    pltpu.sync_copy(b_hbm, bbuf)
    wv = [wbuf[pl.ds(16 * p, 16)] for p in range(P)]
    bval = bbuf[...][0]

    zero = jnp.zeros((16,), jnp.float32)
    ninf = jnp.full((16,), NEG_INF, jnp.float32)
    for k in range(SEG_PER_W):
        for p in range(P):
            acc[k, pl.ds(16 * p, 16)] = zero
            acc[k, pl.ds(D + 16 * p, 16)] = ninf

    io = lax.iota(jnp.int32, 16)
    ta = seg_base + io
    tb = seg_base + 1 + io
    loa = jnp.zeros((16,), jnp.int32)
    lob = jnp.zeros((16,), jnp.int32)

    for stride in (12500, 1563, 196, 25, 3, 1):
        posa = []
        posb = []
        for k in range(8):
            pa = loa + k * stride
            pb = lob + k * stride
            posa.append(pa)
            posb.append(pb)
            idxa[pl.ds(16 * k, 16)] = jnp.minimum(pa, N - 1)
            idxb[pl.ds(16 * k, 16)] = jnp.minimum(pb, N - 1)
        cpa = pltpu.async_copy(ids_hbm.at[idxa], vala, sem)
        cpb = pltpu.async_copy(ids_hbm.at[idxb], valb, sem2)
        cpa.wait()
        cpb.wait()
        ca = jnp.zeros((16,), jnp.int32)
        cb = jnp.zeros((16,), jnp.int32)
        for k in range(8):
            va = vala[pl.ds(16 * k, 16)]
            vb = valb[pl.ds(16 * k, 16)]
            ca = ca + jnp.where((posa[k] < N) & (va < ta), 1, 0)
            cb = cb + jnp.where((posb[k] < N) & (vb < tb), 1, 0)
        loa = jnp.where(ca > 0, loa + (ca - 1) * stride + 1, loa)
        lob = jnp.where(cb > 0, lob + (cb - 1) * stride + 1, lob)

    starts = loa
    ends = lob
    offsa[pl.ds(0, 16)] = starts
    offsb[pl.ds(0, 16)] = ends
    r0 = starts[0]
    r1 = ends[15]

    a0 = (r0 // 8) * 8
    nchunks = lax.div(r1 - a0 + (ROWS - 1), ROWS)
    semf = [semf0, semf1]

    def chunk_base(c):
        return jnp.minimum(a0 + c * ROWS, N - ROWS)

    IW = ROWS + 16

    def issue(c, nb):
        base = chunk_base(c)
        pltpu.async_copy(feats_hbm.at[pl.ds(base, ROWS)],
                         fbuf.at[pl.ds(nb * ROWS, ROWS)], semf[nb])
        pltpu.async_copy(ids_hbm.at[pl.ds(base, ROWS)],
                         idsbuf.at[pl.ds(nb * IW, ROWS)], semf[nb])

    def wait(c, nb):
        base = chunk_base(c)
        pltpu.make_async_copy(feats_hbm.at[pl.ds(base, ROWS)],
                              fbuf.at[pl.ds(nb * ROWS, ROWS)],
                              semf[nb]).wait()
        pltpu.make_async_copy(ids_hbm.at[pl.ds(base, ROWS)],
                              idsbuf.at[pl.ds(nb * IW, ROWS)],
                              semf[nb]).wait()

    def process(c, nb):
        cs = a0 + c * ROWS
        base = chunk_base(c)
        g0 = jnp.maximum(r0, cs)
        g1 = jnp.minimum(cs + ROWS, r1)

        @pl.when(g1 > g0)
        def _process():
            jlo = lax.div(g0 - base, 16)
            jhi = lax.div(g1 - base + 15, 16)

            def gloop(jj, _):
                gv = zero
                for u in range(16):
                    rp = [fbuf[nb * ROWS + 16 * jj + u, pl.ds(16 * p, 16)]
                          for p in range(P)]
                    dv = rp[0] * wv[0]
                    for p in range(1, P):
                        dv = dv + rp[p] * wv[p]
                    dot = lax.reduce_sum_p.bind(dv, axes=(0,)) + bval
                    gv = jnp.where(io == u, jnp.full((16,), dot), gv)
                gatebuf[pl.ds(16 * jj, 16)] = 1.0 / (1.0 + jnp.exp(-gv))
                return 0

            lax.fori_loop(jlo, jhi, gloop, 0)

            kf = idsbuf[pl.ds(nb * IW + g0 - base, 16)][0] - seg_base
            kl = idsbuf[pl.ds(nb * IW + g1 - 1 - base, 16)][0] - seg_base

            def do_seg(k, _):
                s0 = jnp.maximum(offsa[pl.ds(k, 16)][0], g0)
                s1 = jnp.minimum(offsb[pl.ds(k, 16)][0], g1)

                @plsc.parallel_loop(
                    s0 - base, s1 - base, unroll=4,
                    carry=(tuple(zero for _ in range(P)),
                           tuple(ninf for _ in range(P))))
                def rowloop(j, carry):
                    ss, mm = carry
                    g = jnp.full((16,), gatebuf[pl.ds(j, 16)][0],
                                 jnp.float32)
                    rp = [fbuf[nb * ROWS + j, pl.ds(16 * p, 16)]
                          for p in range(P)]
                    ss = tuple(ss[p] + rp[p] * g for p in range(P))
                    mm = tuple(jnp.maximum(mm[p], rp[p]) for p in range(P))
                    return ss, mm

                ss, mm = rowloop
                for p in range(P):
                    plsc.addupdate(acc.at[k, pl.ds(16 * p, 16)], ss[p])
                    mo = acc[k, pl.ds(D + 16 * p, 16)]
                    acc[k, pl.ds(D + 16 * p, 16)] = jnp.maximum(mo, mm[p])
                return 0

            lax.fori_loop(kf, kl + 1, do_seg, 0)

    issue(0, 0)

    def do_pair(pair, _):
        cc = 2 * pair
        for b2 in range(2):
            c = cc + b2

            @pl.when(c < nchunks)
            def _step():
                @pl.when(c + 1 < nchunks)
                def _prefetch():
                    issue(c + 1, (b2 + 1) % 2)

                wait(c, b2)
                process(c, b2)
        return 0

    lax.fori_loop(0, lax.div(nchunks + 1, 2), do_pair, 0)

    pltpu.sync_copy(acc, out_hbm.at[pl.ds(seg_base, SEG_PER_W)])


@jax.jit
def _run(feats, segment_ids, wvec, bpad):
    mesh = plsc.VectorSubcoreMesh(
        core_axis_name="c", subcore_axis_name="s",
        num_cores=NC, num_subcores=NS)
    return pl.kernel(
        _body,
        out_type=jax.ShapeDtypeStruct((B, 2 * D), jnp.float32),
        mesh=mesh,
        compiler_params=pltpu.CompilerParams(needs_layout_passes=False),
        scratch_types=[
            pltpu.VMEM((2 * ROWS, D), jnp.float32),
            pltpu.VMEM((2 * (ROWS + 16),), jnp.int32),
            pltpu.VMEM((SEG_PER_W, 2 * D), jnp.float32),
            pltpu.VMEM((D,), jnp.float32),
            pltpu.VMEM((16,), jnp.float32),
            pltpu.VMEM((128,), jnp.int32),
            pltpu.VMEM((128,), jnp.int32),
            pltpu.VMEM((128,), jnp.int32),
            pltpu.VMEM((128,), jnp.int32),
            pltpu.VMEM((32,), jnp.int32),
            pltpu.VMEM((32,), jnp.int32),
            pltpu.VMEM((ROWS + 16,), jnp.float32),
            pltpu.SemaphoreType.DMA,
            pltpu.SemaphoreType.DMA,
            pltpu.SemaphoreType.DMA,
            pltpu.SemaphoreType.DMA,
        ],
    )(feats, segment_ids, wvec, bpad)


def kernel(feats, segment_ids, W, b):
    wvec = W.reshape(D)
    bpad = jnp.broadcast_to(b.reshape(1), (16,)).astype(jnp.float32)
    return _run(feats, segment_ids.astype(jnp.int32), wvec, bpad)
